# baseline (speedup 1.0000x reference)
import jax
import jax.numpy as jnp
from jax import lax
from jax.experimental import pallas as pl
from jax.experimental.pallas import tpu as pltpu


def _gcn_block_kernel(feat_all_ref, adj_ref, feat_i_ref, w_ref, bias_ref,
                      out_ref):
    agg = jnp.dot(adj_ref[...], feat_all_ref[...],
                  preferred_element_type=jnp.float32)
    x = agg + feat_i_ref[...]
    y = lax.dot_general(x, w_ref[...], (((1,), (1,)), ((), ())),
                        preferred_element_type=jnp.float32)
    out_ref[...] = jnp.maximum(y + bias_ref[...], 0.0)


def kernel(feature, adj, W, bias):
    B, N, D_in = feature.shape
    D_out = W.shape[0]
    feat2 = feature.reshape(N, D_in)
    adj2 = adj.reshape(N, N)
    bias2 = bias.reshape(1, D_out)

    BI = 640
    grid = (pl.cdiv(N, BI),)

    out = pl.pallas_call(
        _gcn_block_kernel,
        grid=grid,
        in_specs=[
            pl.BlockSpec((N, D_in), lambda i: (0, 0)),
            pl.BlockSpec((BI, N), lambda i: (i, 0)),
            pl.BlockSpec((BI, D_in), lambda i: (i, 0)),
            pl.BlockSpec((D_out, D_in), lambda i: (0, 0)),
            pl.BlockSpec((1, D_out), lambda i: (0, 0)),
        ],
        out_specs=pl.BlockSpec((BI, D_out), lambda i: (i, 0)),
        out_shape=jax.ShapeDtypeStruct((N, D_out), jnp.float32),
        compiler_params=pltpu.CompilerParams(
            dimension_semantics=("parallel",),
            vmem_limit_bytes=64 * 1024 * 1024),
    )(feat2, adj2, feat2, W, bias2)
    return out.reshape(B, N, D_out)


# speedup vs baseline: 1.0386x; 1.0386x over previous
import jax
import jax.numpy as jnp
from jax import lax
from jax.experimental import pallas as pl
from jax.experimental.pallas import tpu as pltpu


def _gcn_block_kernel(feat_all_ref, adj_ref, w_ref, bias_ref, out_ref):
    bi = adj_ref.shape[0]
    i = pl.program_id(0)
    agg = jnp.dot(adj_ref[...], feat_all_ref[...],
                  preferred_element_type=jnp.float32)
    x = agg + feat_all_ref[pl.ds(i * bi, bi), :]
    y = lax.dot_general(x, w_ref[...], (((1,), (1,)), ((), ())),
                        preferred_element_type=jnp.float32)
    out_ref[...] = jnp.maximum(y + bias_ref[...], 0.0)


def kernel(feature, adj, W, bias):
    B, N, D_in = feature.shape
    D_out = W.shape[0]
    feat2 = feature.reshape(N, D_in)
    adj2 = adj.reshape(N, N)
    bias2 = bias.reshape(1, D_out)

    BI = 400
    grid = (N // BI,)

    out = pl.pallas_call(
        _gcn_block_kernel,
        grid=grid,
        in_specs=[
            pl.BlockSpec((N, D_in), lambda i: (0, 0)),
            pl.BlockSpec((BI, N), lambda i: (i, 0)),
            pl.BlockSpec((D_out, D_in), lambda i: (0, 0)),
            pl.BlockSpec((1, D_out), lambda i: (0, 0)),
        ],
        out_specs=pl.BlockSpec((BI, D_out), lambda i: (i, 0)),
        out_shape=jax.ShapeDtypeStruct((N, D_out), jnp.float32),
        compiler_params=pltpu.CompilerParams(
            dimension_semantics=("parallel",)),
    )(feat2, adj2, W, bias2)
    return out.reshape(B, N, D_out)
